# baseline (speedup 1.0000x reference)
import jax
import jax.numpy as jnp
from jax.experimental import pallas as pl
from jax.experimental.pallas import tpu as pltpu

_TILE = 1792


def _nn_kernel(x_ref, a_ref, b2_ref, p_ref, out_ref):
    x = x_ref[0]
    m = jax.lax.dot_general(
        x.astype(jnp.bfloat16), a_ref[...].astype(jnp.bfloat16),
        (((1,), (1,)), ((), ())),
        preferred_element_type=jnp.float32,
    )
    a2 = jnp.sum(x * x, axis=1, keepdims=True)
    sq = (a2 + m) + b2_ref[...]
    d = jnp.sqrt(jnp.maximum(sq, 0.0))
    dmin = jnp.min(d, axis=1, keepdims=True)
    iota = jax.lax.broadcasted_iota(jnp.int32, d.shape, 1)
    idx = jnp.min(jnp.where(d == dmin, iota, jnp.int32(1 << 30)), axis=1)
    onehot = (iota == idx[:, None]).astype(jnp.float32)
    out_ref[0] = jax.lax.dot(
        onehot, p_ref[...],
        precision=jax.lax.Precision.HIGHEST,
        preferred_element_type=jnp.float32,
    )


def kernel(x, palette):
    B, C, H, W = x.shape
    HW = H * W
    K = palette.shape[0]
    xt = x.reshape(B, C, HW).transpose(0, 2, 1)
    b2 = jnp.sum(palette * palette, axis=1)[None, :]
    a = -2.0 * palette
    nt = HW // _TILE
    out = pl.pallas_call(
        _nn_kernel,
        grid=(B, nt),
        in_specs=[
            pl.BlockSpec((1, _TILE, C), lambda b, t: (b, t, 0)),
            pl.BlockSpec((K, C), lambda b, t: (0, 0)),
            pl.BlockSpec((1, K), lambda b, t: (0, 0)),
            pl.BlockSpec((K, C), lambda b, t: (0, 0)),
        ],
        out_specs=pl.BlockSpec((1, _TILE, C), lambda b, t: (b, t, 0)),
        out_shape=jax.ShapeDtypeStruct((B, HW, C), x.dtype),
        compiler_params=pltpu.CompilerParams(
            dimension_semantics=("parallel", "parallel"),
        ),
    )(xt, a, b2, palette)
    return out.transpose(0, 2, 1).reshape(B, C, H, W)


# speedup vs baseline: 1.0641x; 1.0641x over previous
import jax
import jax.numpy as jnp
from jax.experimental import pallas as pl
from jax.experimental.pallas import tpu as pltpu

_TILE = 1792


def _nn_kernel(x_ref, a_ref, b2_ref, fi_ref, p_ref, out_ref):
    x = x_ref[0]
    m = jax.lax.dot_general(
        x.astype(jnp.bfloat16), a_ref[...].astype(jnp.bfloat16),
        (((1,), (1,)), ((), ())),
        preferred_element_type=jnp.float32,
    )
    a2 = jnp.sum(x * x, axis=1, keepdims=True)
    sq = (a2 + m) + b2_ref[...]
    d = jnp.sqrt(jnp.maximum(sq, 0.0))
    dmin = jnp.min(d, axis=1, keepdims=True)
    masked = jnp.where(d == dmin, fi_ref[...], jnp.float32(512.0))
    idxf = jnp.min(masked, axis=1, keepdims=True)
    onehot = (masked == idxf).astype(jnp.float32)
    out_ref[0] = jax.lax.dot(
        onehot, p_ref[...],
        precision=jax.lax.Precision.HIGHEST,
        preferred_element_type=jnp.float32,
    )


def kernel(x, palette):
    B, C, H, W = x.shape
    HW = H * W
    K = palette.shape[0]
    xt = x.reshape(B, C, HW).transpose(0, 2, 1)
    b2 = jnp.sum(palette * palette, axis=1)[None, :]
    a = -2.0 * palette
    fiota = jnp.arange(K, dtype=jnp.float32)[None, :]
    nt = HW // _TILE
    out = pl.pallas_call(
        _nn_kernel,
        grid=(B, nt),
        in_specs=[
            pl.BlockSpec((1, _TILE, C), lambda b, t: (b, t, 0)),
            pl.BlockSpec((K, C), lambda b, t: (0, 0)),
            pl.BlockSpec((1, K), lambda b, t: (0, 0)),
            pl.BlockSpec((1, K), lambda b, t: (0, 0)),
            pl.BlockSpec((K, C), lambda b, t: (0, 0)),
        ],
        out_specs=pl.BlockSpec((1, _TILE, C), lambda b, t: (b, t, 0)),
        out_shape=jax.ShapeDtypeStruct((B, HW, C), x.dtype),
        compiler_params=pltpu.CompilerParams(
            dimension_semantics=("parallel", "parallel"),
        ),
    )(xt, a, b2, fiota, palette)
    return out.transpose(0, 2, 1).reshape(B, C, H, W)


# speedup vs baseline: 2.2821x; 2.1446x over previous
import jax
import jax.numpy as jnp
from jax.experimental import pallas as pl
from jax.experimental.pallas import tpu as pltpu

_TILE = 1792


def _nn_kernel(x_ref, a_ref, b2_ref, fi_ref, p1_ref, p2_ref, p3_ref, out_ref):
    x = x_ref[0]
    m = jax.lax.dot(
        a_ref[...].astype(jnp.bfloat16), x.astype(jnp.bfloat16),
        preferred_element_type=jnp.float32,
    )
    a2 = jnp.sum(x * x, axis=0, keepdims=True)
    sq = (a2 + m) + b2_ref[...]
    d = jnp.sqrt(jnp.maximum(sq, 0.0))
    dmin = jnp.min(d, axis=0, keepdims=True)
    masked = jnp.where(d == dmin, fi_ref[...], jnp.float32(512.0))
    idxf = jnp.min(masked, axis=0, keepdims=True)
    onehot = (masked == idxf).astype(jnp.bfloat16)
    g1 = jax.lax.dot(p1_ref[...], onehot, preferred_element_type=jnp.float32)
    g2 = jax.lax.dot(p2_ref[...], onehot, preferred_element_type=jnp.float32)
    g3 = jax.lax.dot(p3_ref[...], onehot, preferred_element_type=jnp.float32)
    out_ref[0] = g1 + (g2 + g3)


def kernel(x, palette):
    B, C, H, W = x.shape
    HW = H * W
    K = palette.shape[0]
    xr = x.reshape(B, C, HW)
    b2 = jnp.sum(palette * palette, axis=1)
    b2f = jnp.broadcast_to(b2[:, None], (K, _TILE))
    fif = jnp.broadcast_to(
        jnp.arange(K, dtype=jnp.float32)[:, None], (K, _TILE)
    )
    a = -2.0 * palette
    pt = palette.T
    p1 = pt.astype(jnp.bfloat16)
    r1 = pt - p1.astype(jnp.float32)
    p2 = r1.astype(jnp.bfloat16)
    p3 = (r1 - p2.astype(jnp.float32)).astype(jnp.bfloat16)
    nt = HW // _TILE
    out = pl.pallas_call(
        _nn_kernel,
        grid=(B, nt),
        in_specs=[
            pl.BlockSpec((1, C, _TILE), lambda b, t: (b, 0, t)),
            pl.BlockSpec((K, C), lambda b, t: (0, 0)),
            pl.BlockSpec((K, _TILE), lambda b, t: (0, 0)),
            pl.BlockSpec((K, _TILE), lambda b, t: (0, 0)),
            pl.BlockSpec((C, K), lambda b, t: (0, 0)),
            pl.BlockSpec((C, K), lambda b, t: (0, 0)),
            pl.BlockSpec((C, K), lambda b, t: (0, 0)),
        ],
        out_specs=pl.BlockSpec((1, C, _TILE), lambda b, t: (b, 0, t)),
        out_shape=jax.ShapeDtypeStruct((B, C, HW), x.dtype),
        compiler_params=pltpu.CompilerParams(
            dimension_semantics=("parallel", "parallel"),
        ),
    )(xr, a, b2f, fif, p1, p2, p3)
    return out.reshape(B, C, H, W)


# speedup vs baseline: 2.7374x; 1.1995x over previous
import jax
import jax.numpy as jnp
from jax.experimental import pallas as pl
from jax.experimental.pallas import tpu as pltpu

_TILE = 1792


def _nn_kernel(x_ref, a_ref, b2_ref, fi_ref, ps_ref, out_ref):
    x = x_ref[0]
    m = jax.lax.dot(
        a_ref[...].astype(jnp.bfloat16), x.astype(jnp.bfloat16),
        preferred_element_type=jnp.float32,
    )
    x0, x1, x2 = x[0:1, :], x[1:2, :], x[2:3, :]
    a2 = (x0 * x0 + x1 * x1) + x2 * x2
    sq = (a2 + m) + b2_ref[...]
    d = jnp.sqrt(jnp.maximum(sq, 0.0))
    dmin = jnp.min(d, axis=0, keepdims=True)
    masked = jnp.where(d == dmin, fi_ref[...], jnp.float32(512.0))
    idxf = jnp.min(masked, axis=0, keepdims=True)
    onehot = (masked == idxf).astype(jnp.bfloat16)
    g = jax.lax.dot(ps_ref[...], onehot, preferred_element_type=jnp.float32)
    out_ref[0] = g[0:3] + (g[3:6] + g[6:9])


def kernel(x, palette):
    B, C, H, W = x.shape
    HW = H * W
    K = palette.shape[0]
    xr = x.reshape(B, C, HW)
    b2 = jnp.sum(palette * palette, axis=1)
    b2f = jnp.broadcast_to(b2[:, None], (K, _TILE))
    fif = jnp.broadcast_to(
        jnp.arange(K, dtype=jnp.float32)[:, None], (K, _TILE)
    )
    a = -2.0 * palette
    pt = palette.T
    p1 = pt.astype(jnp.bfloat16)
    r1 = pt - p1.astype(jnp.float32)
    p2 = r1.astype(jnp.bfloat16)
    p3 = (r1 - p2.astype(jnp.float32)).astype(jnp.bfloat16)
    ps = jnp.concatenate([p1, p2, p3], axis=0)
    nt = HW // _TILE
    out = pl.pallas_call(
        _nn_kernel,
        grid=(B, nt),
        in_specs=[
            pl.BlockSpec((1, C, _TILE), lambda b, t: (b, 0, t)),
            pl.BlockSpec((K, C), lambda b, t: (0, 0)),
            pl.BlockSpec((K, _TILE), lambda b, t: (0, 0)),
            pl.BlockSpec((K, _TILE), lambda b, t: (0, 0)),
            pl.BlockSpec((3 * C, K), lambda b, t: (0, 0)),
        ],
        out_specs=pl.BlockSpec((1, C, _TILE), lambda b, t: (b, 0, t)),
        out_shape=jax.ShapeDtypeStruct((B, C, HW), x.dtype),
        compiler_params=pltpu.CompilerParams(
            dimension_semantics=("parallel", "parallel"),
        ),
    )(xr, a, b2f, fif, ps)
    return out.reshape(B, C, H, W)


# speedup vs baseline: 3.6661x; 1.3392x over previous
import jax
import jax.numpy as jnp
from jax.experimental import pallas as pl
from jax.experimental.pallas import tpu as pltpu

_TILE = 1792


def _nn_kernel(x_ref, a_ref, b2_ref, fi_ref, ps_ref, out_ref):
    x = x_ref[0]
    m = jax.lax.dot(
        a_ref[...].astype(jnp.bfloat16), x.astype(jnp.bfloat16),
        preferred_element_type=jnp.float32,
    )
    x0, x1, x2 = x[0:1, :], x[1:2, :], x[2:3, :]
    a2 = (x0 * x0 + x1 * x1) + x2 * x2
    sq = (a2 + m) + b2_ref[...]
    sqmin = jnp.min(sq, axis=0, keepdims=True)
    dmin = jnp.sqrt(jnp.maximum(sqmin, 0.0))
    y2 = dmin * dmin
    bits = jax.lax.bitcast_convert_type(y2, jnp.int32)
    xub = jnp.maximum(sqmin, 0.0)
    for i in range(-4, 5):
        c = jax.lax.bitcast_convert_type(
            jnp.maximum(bits + i, 0), jnp.float32
        )
        xub = jnp.where(jnp.sqrt(c) <= dmin, c, xub)
    masked = jnp.where(sq <= xub, fi_ref[...], jnp.float32(512.0))
    idxf = jnp.min(masked, axis=0, keepdims=True)
    onehot = (masked == idxf).astype(jnp.bfloat16)
    g = jax.lax.dot(ps_ref[...], onehot, preferred_element_type=jnp.float32)
    out_ref[0] = g[0:3] + (g[3:6] + g[6:9])


def kernel(x, palette):
    B, C, H, W = x.shape
    HW = H * W
    K = palette.shape[0]
    xr = x.reshape(B, C, HW)
    b2 = jnp.sum(palette * palette, axis=1)
    b2f = jnp.broadcast_to(b2[:, None], (K, _TILE))
    fif = jnp.broadcast_to(
        jnp.arange(K, dtype=jnp.float32)[:, None], (K, _TILE)
    )
    a = -2.0 * palette
    pt = palette.T
    p1 = pt.astype(jnp.bfloat16)
    r1 = pt - p1.astype(jnp.float32)
    p2 = r1.astype(jnp.bfloat16)
    p3 = (r1 - p2.astype(jnp.float32)).astype(jnp.bfloat16)
    ps = jnp.concatenate([p1, p2, p3], axis=0)
    nt = HW // _TILE
    out = pl.pallas_call(
        _nn_kernel,
        grid=(B, nt),
        in_specs=[
            pl.BlockSpec((1, C, _TILE), lambda b, t: (b, 0, t)),
            pl.BlockSpec((K, C), lambda b, t: (0, 0)),
            pl.BlockSpec((K, _TILE), lambda b, t: (0, 0)),
            pl.BlockSpec((K, _TILE), lambda b, t: (0, 0)),
            pl.BlockSpec((3 * C, K), lambda b, t: (0, 0)),
        ],
        out_specs=pl.BlockSpec((1, C, _TILE), lambda b, t: (b, 0, t)),
        out_shape=jax.ShapeDtypeStruct((B, C, HW), x.dtype),
        compiler_params=pltpu.CompilerParams(
            dimension_semantics=("parallel", "parallel"),
        ),
    )(xr, a, b2f, fif, ps)
    return out.reshape(B, C, H, W)
